# Initial kernel scaffold; baseline (speedup 1.0000x reference)
import jax
import jax.numpy as jnp
from jax.experimental import pallas as pl


def kernel(points, features, params):
    raise NotImplementedError("write your pallas kernel here")


# speedup vs baseline: 6.1733x; 6.1733x over previous
import functools
import jax
import jax.numpy as jnp
from jax.experimental import pallas as pl
from jax.experimental.pallas import tpu as pltpu

_EPS = 1e-5
_N = 100
_NP = 104
_K = 7
_BIG = 1e10


def _fold_lin(W, bb, g, be):
    s = g * (1.0 / jnp.sqrt(1.0 + _EPS))
    return W * s[:, None], bb * s + be


def _dotT(a, b):
    return jax.lax.dot_general(a, b, (((1,), (1,)), ((), ())),
                               preferred_element_type=jnp.float32)


def _dot(a, b):
    return jnp.dot(a, b, preferred_element_type=jnp.float32)


def _knn_onehots(pos):
    gram = _dotT(pos, pos)
    rowi = jax.lax.broadcasted_iota(jnp.int32, (_NP, _NP), 0)
    coli = jax.lax.broadcasted_iota(jnp.int32, (_NP, _NP), 1)
    eye = (rowi == coli).astype(jnp.float32)
    diag_col = jnp.sum(gram * eye, axis=1, keepdims=True)
    diag_row = jnp.sum(gram * eye, axis=0, keepdims=True)
    d2 = diag_col - 2.0 * gram + diag_row
    invalid = (rowi == coli) | (coli >= _N)
    d2 = jnp.where(invalid, _BIG, d2)
    ohs = []
    for _ in range(_K):
        m = jnp.min(d2, axis=1, keepdims=True)
        cand = jnp.where(d2 <= m, coli, jnp.int32(2 * _NP))
        jstar = jnp.min(cand, axis=1, keepdims=True)
        oh = coli == jstar
        ohs.append(oh.astype(jnp.float32))
        d2 = jnp.where(oh, _BIG, d2)
    return jnp.concatenate(ohs, axis=0)


def _edge_conv(xin, oh, WA, WB, b1, W2, b2, W3, b3, Wsc, bsc):
    A = _dot(xin, WA) + b1
    Bm = _dot(xin, WB)
    G = _dot(oh, Bm)
    At = jnp.concatenate([A] * _K, axis=0)
    H1 = jax.nn.relu(At + G)
    H2 = jax.nn.relu(_dot(H1, W2) + b2)
    H3 = jax.nn.relu(_dot(H2, W3) + b3)
    agg = H3[0:_NP]
    for t in range(1, _K):
        agg = agg + H3[t * _NP:(t + 1) * _NP]
    agg = agg * (1.0 / _K)
    scv = _dot(xin, Wsc) + bsc
    return jax.nn.relu(agg + scv)


def _jet_kernel(x_ref, c_ref, s0_ref, b0_ref,
                WA1_ref, WB1_ref, b11_ref, W12_ref, b12_ref, W13_ref, b13_ref,
                Wsc1_ref, bsc1_ref,
                WA2_ref, WB2_ref, b21_ref, W22_ref, b22_ref, W23_ref, b23_ref,
                Wsc2_ref, bsc2_ref,
                Wf1_ref, Wf2_ref, bf_ref, Wg1_ref, bg1_ref, Wg2_ref, bg2_ref,
                out_ref):
    x = x_ref[0]
    c = c_ref[0]
    xbn = x * s0_ref[...] + b0_ref[...]

    oh1 = _knn_onehots(c)
    x1 = _edge_conv(xbn, oh1, WA1_ref[...], WB1_ref[...], b11_ref[...],
                    W12_ref[...], b12_ref[...], W13_ref[...], b13_ref[...],
                    Wsc1_ref[...], bsc1_ref[...])

    oh2 = _knn_onehots(x1)
    x2 = _edge_conv(x1, oh2, WA2_ref[...], WB2_ref[...], b21_ref[...],
                    W22_ref[...], b22_ref[...], W23_ref[...], b23_ref[...],
                    Wsc2_ref[...], bsc2_ref[...])

    f = jax.nn.relu(_dot(x1, Wf1_ref[...]) + _dot(x2, Wf2_ref[...])
                    + bf_ref[...])
    rmask = jax.lax.broadcasted_iota(jnp.int32, (_NP, 1), 0) < _N
    pooled = jnp.sum(jnp.where(rmask, f, 0.0), axis=0, keepdims=True) / _N
    h = jax.nn.relu(_dot(pooled, Wg1_ref[...]) + bg1_ref[...])
    out = _dot(h, Wg2_ref[...]) + bg2_ref[...]
    out_ref[0] = out


@jax.jit
def kernel(points, features, params):
    B, _, N = points.shape
    F = features.shape[1]

    g0, be0 = params['bn_fts']
    s0 = (g0 / jnp.sqrt(1.0 + _EPS)).reshape(1, F)
    b0 = be0.reshape(1, F)

    def conv_params(mlp, sc, fin):
        (W1, bb1, g1, be1), (W2, bb2, g2, be2), (W3, bb3, g3, be3) = mlp
        W1f, b1f = _fold_lin(W1, bb1, g1, be1)
        Wa, Wb = W1f[:, :fin], W1f[:, fin:]
        W2f, b2f = _fold_lin(W2, bb2, g2, be2)
        W3f, b3f = _fold_lin(W3, bb3, g3, be3)
        Wsc, gsc, bsc = sc
        Wscf = Wsc * (gsc / jnp.sqrt(1.0 + _EPS))[:, None]
        h = W1f.shape[0]
        return [(Wa - Wb).T, Wb.T, b1f.reshape(1, h),
                W2f.T, b2f.reshape(1, h), W3f.T, b3f.reshape(1, h),
                Wscf.T, bsc.reshape(1, h)]

    p1 = conv_params(params['conv1_mlp'], params['conv1_sc'], F)
    p2 = conv_params(params['conv2_mlp'], params['conv2_sc'], 32)

    Wf, gf, bf = params['fusion']
    Wff = (Wf * (gf / jnp.sqrt(1.0 + _EPS))[:, None]).T
    Wf1, Wf2 = Wff[:32], Wff[32:]
    bff = bf.reshape(1, -1)

    Wg1, bg1 = params['fc1']
    Wg1T = Wg1.T
    bg1 = bg1.reshape(1, -1)
    Wg2, bg2 = params['fc2']
    ncls = Wg2.shape[0]
    Wg2T = jnp.pad(Wg2.T, ((0, 0), (0, 128 - ncls)))
    bg2p = jnp.pad(bg2.reshape(1, -1), ((0, 0), (0, 128 - ncls)))

    xT = jnp.transpose(features, (0, 2, 1))
    xT = jnp.pad(xT, ((0, 0), (0, _NP - N), (0, 0)))
    cT = jnp.transpose(points, (0, 2, 1))
    cT = jnp.pad(cT, ((0, 0), (0, _NP - N), (0, F - 2)))

    weights = [s0, b0] + p1 + p2 + [Wf1, Wf2, bff, Wg1T, bg1, Wg2T, bg2p]

    def wspec(w):
        nd = w.ndim
        return pl.BlockSpec(w.shape, lambda i, _nd=nd: (0,) * _nd)

    out = pl.pallas_call(
        _jet_kernel,
        grid=(B,),
        in_specs=[pl.BlockSpec((1, _NP, F), lambda i: (i, 0, 0)),
                  pl.BlockSpec((1, _NP, F), lambda i: (i, 0, 0))]
                 + [wspec(w) for w in weights],
        out_specs=pl.BlockSpec((1, 1, 128), lambda i: (i, 0, 0)),
        out_shape=jax.ShapeDtypeStruct((B, 1, 128), jnp.float32),
    )(xT, cT, *weights)
    return out[:, 0, :ncls]


# speedup vs baseline: 9.9115x; 1.6055x over previous
import jax
import jax.numpy as jnp
from jax.experimental import pallas as pl

_EPS = 1e-5
_N = 100
_NP = 104
_K = 7
_G = 4
_BIG = 1e10


def _fold_lin(W, bb, g, be):
    s = g * (1.0 / jnp.sqrt(1.0 + _EPS))
    return W * s[:, None], bb * s + be


def _dotT(a, b):
    return jax.lax.dot_general(a, b, (((1,), (1,)), ((), ())),
                               preferred_element_type=jnp.float32)


def _dot(a, b):
    return jnp.dot(a, b, preferred_element_type=jnp.float32)


def _knn_onehots(d2, coli_f):
    ohs = []
    for _ in range(_K):
        m = jnp.min(d2, axis=1, keepdims=True)
        cand = jnp.where(d2 <= m, coli_f, 3.0 * _NP)
        jstar = jnp.min(cand, axis=1, keepdims=True)
        oh = (coli_f == jstar).astype(jnp.float32)
        ohs.append(oh)
        d2 = d2 + oh * _BIG
    return jnp.concatenate(ohs, axis=0)


def _edge_conv(xin, oh, WA, WB, b1, W2, b2, W3, b3, Wsc, bsc):
    A = _dot(xin, WA) + b1
    Bm = _dot(xin, WB)
    G = _dot(oh, Bm)
    At = jnp.concatenate([A] * _K, axis=0)
    H1 = jax.nn.relu(At + G)
    H2 = jax.nn.relu(_dot(H1, W2) + b2)
    H3 = jax.nn.relu(_dot(H2, W3) + b3)
    agg = H3[0:_NP]
    for t in range(1, _K):
        agg = agg + H3[t * _NP:(t + 1) * _NP]
    agg = agg * (1.0 / _K)
    scv = _dot(xin, Wsc) + bsc
    return jax.nn.relu(agg + scv)


def _jet_kernel(x_ref, ccol_ref, crow_ref, s0_ref, b0_ref,
                WA1_ref, WB1_ref, b11_ref, W12_ref, b12_ref, W13_ref, b13_ref,
                Wsc1_ref, bsc1_ref,
                WA2_ref, WB2_ref, b21_ref, W22_ref, b22_ref, W23_ref, b23_ref,
                Wsc2_ref, bsc2_ref,
                Wf1_ref, Wf2_ref, bf_ref, Wg1_ref, bg1_ref, Wg2_ref, bg2_ref,
                out_ref):
    rowi = jax.lax.broadcasted_iota(jnp.int32, (_NP, _NP), 0)
    coli = jax.lax.broadcasted_iota(jnp.int32, (_NP, _NP), 1)
    coli_f = coli.astype(jnp.float32)
    invalid = (rowi == coli) | (coli >= _N)
    eye = (rowi == coli).astype(jnp.float32)
    rmask = jax.lax.broadcasted_iota(jnp.int32, (_NP, 1), 0) < _N

    for g in range(_G):
        x = x_ref[g]
        xbn = x * s0_ref[...] + b0_ref[...]

        cxr = crow_ref[g, 0:1, 0:_NP]
        cyr = crow_ref[g, 1:2, 0:_NP]
        cxc = ccol_ref[g, :, 0:1]
        cyc = ccol_ref[g, :, 1:2]
        dx = cxc - cxr
        dy = cyc - cyr
        d2 = jnp.where(invalid, _BIG, dx * dx + dy * dy)
        oh1 = _knn_onehots(d2, coli_f)
        x1 = _edge_conv(xbn, oh1, WA1_ref[...], WB1_ref[...], b11_ref[...],
                        W12_ref[...], b12_ref[...], W13_ref[...], b13_ref[...],
                        Wsc1_ref[...], bsc1_ref[...])

        gram = _dotT(x1, x1)
        diag_c = jnp.sum(x1 * x1, axis=1, keepdims=True)
        diag_r = jnp.sum(gram * eye, axis=0, keepdims=True)
        d2b = diag_c - 2.0 * gram + diag_r
        d2b = jnp.where(invalid, _BIG, d2b)
        oh2 = _knn_onehots(d2b, coli_f)
        x2 = _edge_conv(x1, oh2, WA2_ref[...], WB2_ref[...], b21_ref[...],
                        W22_ref[...], b22_ref[...], W23_ref[...], b23_ref[...],
                        Wsc2_ref[...], bsc2_ref[...])

        f = jax.nn.relu(_dot(x1, Wf1_ref[...]) + _dot(x2, Wf2_ref[...])
                        + bf_ref[...])
        pooled = jnp.sum(jnp.where(rmask, f, 0.0), axis=0, keepdims=True) / _N
        h = jax.nn.relu(_dot(pooled, Wg1_ref[...]) + bg1_ref[...])
        out_ref[g] = _dot(h, Wg2_ref[...]) + bg2_ref[...]


@jax.jit
def kernel(points, features, params):
    B, _, N = points.shape
    F = features.shape[1]

    g0, be0 = params['bn_fts']
    s0 = (g0 / jnp.sqrt(1.0 + _EPS)).reshape(1, F)
    b0 = be0.reshape(1, F)

    def conv_params(mlp, sc, fin):
        (W1, bb1, g1, be1), (W2, bb2, g2, be2), (W3, bb3, g3, be3) = mlp
        W1f, b1f = _fold_lin(W1, bb1, g1, be1)
        Wa, Wb = W1f[:, :fin], W1f[:, fin:]
        W2f, b2f = _fold_lin(W2, bb2, g2, be2)
        W3f, b3f = _fold_lin(W3, bb3, g3, be3)
        Wsc, gsc, bsc = sc
        Wscf = Wsc * (gsc / jnp.sqrt(1.0 + _EPS))[:, None]
        h = W1f.shape[0]
        return [(Wa - Wb).T, Wb.T, b1f.reshape(1, h),
                W2f.T, b2f.reshape(1, h), W3f.T, b3f.reshape(1, h),
                Wscf.T, bsc.reshape(1, h)]

    p1 = conv_params(params['conv1_mlp'], params['conv1_sc'], F)
    p2 = conv_params(params['conv2_mlp'], params['conv2_sc'], 32)

    Wf, gf, bf = params['fusion']
    Wff = (Wf * (gf / jnp.sqrt(1.0 + _EPS))[:, None]).T
    Wf1, Wf2 = Wff[:32], Wff[32:]
    bff = bf.reshape(1, -1)

    Wg1, bg1 = params['fc1']
    Wg1T = Wg1.T
    bg1 = bg1.reshape(1, -1)
    Wg2, bg2 = params['fc2']
    ncls = Wg2.shape[0]
    Wg2T = jnp.pad(Wg2.T, ((0, 0), (0, 128 - ncls)))
    bg2p = jnp.pad(bg2.reshape(1, -1), ((0, 0), (0, 128 - ncls)))

    xT = jnp.transpose(features, (0, 2, 1))
    xT = jnp.pad(xT, ((0, 0), (0, _NP - N), (0, 0)))
    cT = jnp.transpose(points, (0, 2, 1))
    ccol = jnp.pad(cT, ((0, 0), (0, _NP - N), (0, F - 2)))
    crow = jnp.pad(points, ((0, 0), (0, 8 - 2), (0, F - N)))

    weights = [s0, b0] + p1 + p2 + [Wf1, Wf2, bff, Wg1T, bg1, Wg2T, bg2p]

    def wspec(w):
        nd = w.ndim
        return pl.BlockSpec(w.shape, lambda i, _nd=nd: (0,) * _nd)

    out = pl.pallas_call(
        _jet_kernel,
        grid=(B // _G,),
        in_specs=[pl.BlockSpec((_G, _NP, F), lambda i: (i, 0, 0)),
                  pl.BlockSpec((_G, _NP, F), lambda i: (i, 0, 0)),
                  pl.BlockSpec((_G, 8, F), lambda i: (i, 0, 0))]
                 + [wspec(w) for w in weights],
        out_specs=pl.BlockSpec((_G, 1, 128), lambda i: (i, 0, 0)),
        out_shape=jax.ShapeDtypeStruct((B, 1, 128), jnp.float32),
    )(xT, ccol, crow, *weights)
    return out[:, 0, :ncls]


# speedup vs baseline: 10.2356x; 1.0327x over previous
import jax
import jax.numpy as jnp
from jax.experimental import pallas as pl

_EPS = 1e-5
_N = 100
_NP = 104
_K = 7
_G = 8
_BIG = 1e10


def _fold_lin(W, bb, g, be):
    s = g * (1.0 / jnp.sqrt(1.0 + _EPS))
    return W * s[:, None], bb * s + be


def _dotT(a, b):
    return jax.lax.dot_general(a, b, (((1,), (1,)), ((), ())),
                               preferred_element_type=jnp.float32)


def _dot(a, b):
    return jnp.dot(a, b, preferred_element_type=jnp.float32)


def _knn_onehots(d2, coli_f):
    ohs = []
    for _ in range(_K):
        m = jnp.min(d2, axis=1, keepdims=True)
        cand = jnp.where(d2 <= m, coli_f, 3.0 * _NP)
        jstar = jnp.min(cand, axis=1, keepdims=True)
        oh = (coli_f == jstar).astype(jnp.float32)
        ohs.append(oh)
        d2 = d2 + oh * _BIG
    return jnp.concatenate(ohs, axis=0)


def _edge_conv(xin, oh, WA, WB, b1, W2, b2, W3, b3, Wsc, bsc):
    A = _dot(xin, WA) + b1
    Bm = _dot(xin, WB)
    G = _dot(oh, Bm)
    At = jnp.concatenate([A] * _K, axis=0)
    H1 = jax.nn.relu(At + G)
    H2 = jax.nn.relu(_dot(H1, W2) + b2)
    H3 = jax.nn.relu(_dot(H2, W3) + b3)
    agg = H3[0:_NP]
    for t in range(1, _K):
        agg = agg + H3[t * _NP:(t + 1) * _NP]
    agg = agg * (1.0 / _K)
    scv = _dot(xin, Wsc) + bsc
    return jax.nn.relu(agg + scv)


def _jet_kernel(x_ref, ccol_ref, crow_ref, s0_ref, b0_ref,
                WA1_ref, WB1_ref, b11_ref, W12_ref, b12_ref, W13_ref, b13_ref,
                Wsc1_ref, bsc1_ref,
                WA2_ref, WB2_ref, b21_ref, W22_ref, b22_ref, W23_ref, b23_ref,
                Wsc2_ref, bsc2_ref,
                Wf1_ref, Wf2_ref, bf_ref, Wg1_ref, bg1_ref, Wg2_ref, bg2_ref,
                out_ref):
    rowi = jax.lax.broadcasted_iota(jnp.int32, (_NP, _NP), 0)
    coli = jax.lax.broadcasted_iota(jnp.int32, (_NP, _NP), 1)
    coli_f = coli.astype(jnp.float32)
    invalid = (rowi == coli) | (coli >= _N)
    eye = (rowi == coli).astype(jnp.float32)
    rmask = jax.lax.broadcasted_iota(jnp.int32, (_NP, 1), 0) < _N

    for g in range(_G):
        x = x_ref[g]
        xbn = x * s0_ref[...] + b0_ref[...]

        cxr = crow_ref[g, 0:1, 0:_NP]
        cyr = crow_ref[g, 1:2, 0:_NP]
        cxc = ccol_ref[g, :, 0:1]
        cyc = ccol_ref[g, :, 1:2]
        dx = cxc - cxr
        dy = cyc - cyr
        d2 = jnp.where(invalid, _BIG, dx * dx + dy * dy)
        oh1 = _knn_onehots(d2, coli_f)
        x1 = _edge_conv(xbn, oh1, WA1_ref[...], WB1_ref[...], b11_ref[...],
                        W12_ref[...], b12_ref[...], W13_ref[...], b13_ref[...],
                        Wsc1_ref[...], bsc1_ref[...])

        gram = _dotT(x1, x1)
        diag_c = jnp.sum(x1 * x1, axis=1, keepdims=True)
        diag_r = jnp.sum(gram * eye, axis=0, keepdims=True)
        d2b = diag_c - 2.0 * gram + diag_r
        d2b = jnp.where(invalid, _BIG, d2b)
        oh2 = _knn_onehots(d2b, coli_f)
        x2 = _edge_conv(x1, oh2, WA2_ref[...], WB2_ref[...], b21_ref[...],
                        W22_ref[...], b22_ref[...], W23_ref[...], b23_ref[...],
                        Wsc2_ref[...], bsc2_ref[...])

        f = jax.nn.relu(_dot(x1, Wf1_ref[...]) + _dot(x2, Wf2_ref[...])
                        + bf_ref[...])
        pooled = jnp.sum(jnp.where(rmask, f, 0.0), axis=0, keepdims=True) / _N
        h = jax.nn.relu(_dot(pooled, Wg1_ref[...]) + bg1_ref[...])
        out_ref[g] = _dot(h, Wg2_ref[...]) + bg2_ref[...]


@jax.jit
def kernel(points, features, params):
    B, _, N = points.shape
    F = features.shape[1]

    g0, be0 = params['bn_fts']
    s0 = (g0 / jnp.sqrt(1.0 + _EPS)).reshape(1, F)
    b0 = be0.reshape(1, F)

    def conv_params(mlp, sc, fin):
        (W1, bb1, g1, be1), (W2, bb2, g2, be2), (W3, bb3, g3, be3) = mlp
        W1f, b1f = _fold_lin(W1, bb1, g1, be1)
        Wa, Wb = W1f[:, :fin], W1f[:, fin:]
        W2f, b2f = _fold_lin(W2, bb2, g2, be2)
        W3f, b3f = _fold_lin(W3, bb3, g3, be3)
        Wsc, gsc, bsc = sc
        Wscf = Wsc * (gsc / jnp.sqrt(1.0 + _EPS))[:, None]
        h = W1f.shape[0]
        return [(Wa - Wb).T, Wb.T, b1f.reshape(1, h),
                W2f.T, b2f.reshape(1, h), W3f.T, b3f.reshape(1, h),
                Wscf.T, bsc.reshape(1, h)]

    p1 = conv_params(params['conv1_mlp'], params['conv1_sc'], F)
    p2 = conv_params(params['conv2_mlp'], params['conv2_sc'], 32)

    Wf, gf, bf = params['fusion']
    Wff = (Wf * (gf / jnp.sqrt(1.0 + _EPS))[:, None]).T
    Wf1, Wf2 = Wff[:32], Wff[32:]
    bff = bf.reshape(1, -1)

    Wg1, bg1 = params['fc1']
    Wg1T = Wg1.T
    bg1 = bg1.reshape(1, -1)
    Wg2, bg2 = params['fc2']
    ncls = Wg2.shape[0]
    Wg2T = jnp.pad(Wg2.T, ((0, 0), (0, 128 - ncls)))
    bg2p = jnp.pad(bg2.reshape(1, -1), ((0, 0), (0, 128 - ncls)))

    xT = jnp.transpose(features, (0, 2, 1))
    xT = jnp.pad(xT, ((0, 0), (0, _NP - N), (0, 0)))
    cT = jnp.transpose(points, (0, 2, 1))
    ccol = jnp.pad(cT, ((0, 0), (0, _NP - N), (0, F - 2)))
    crow = jnp.pad(points, ((0, 0), (0, 8 - 2), (0, F - N)))

    weights = [s0, b0] + p1 + p2 + [Wf1, Wf2, bff, Wg1T, bg1, Wg2T, bg2p]

    def wspec(w):
        nd = w.ndim
        return pl.BlockSpec(w.shape, lambda i, _nd=nd: (0,) * _nd)

    out = pl.pallas_call(
        _jet_kernel,
        grid=(B // _G,),
        in_specs=[pl.BlockSpec((_G, _NP, F), lambda i: (i, 0, 0)),
                  pl.BlockSpec((_G, _NP, F), lambda i: (i, 0, 0)),
                  pl.BlockSpec((_G, 8, F), lambda i: (i, 0, 0))]
                 + [wspec(w) for w in weights],
        out_specs=pl.BlockSpec((_G, 1, 128), lambda i: (i, 0, 0)),
        out_shape=jax.ShapeDtypeStruct((B, 1, 128), jnp.float32),
    )(xT, ccol, crow, *weights)
    return out[:, 0, :ncls]


# speedup vs baseline: 20.4004x; 1.9931x over previous
import jax
import jax.numpy as jnp
from jax.experimental import pallas as pl

_EPS = 1e-5
_N = 100
_NP = 104
_K = 7
_G = 8
_BIG = 1e10


def _fold_lin(W, bb, g, be):
    s = g * (1.0 / jnp.sqrt(1.0 + _EPS))
    return W * s[:, None], bb * s + be


def _dotT(a, b):
    return jax.lax.dot_general(a, b, (((1,), (1,)), ((), ())),
                               preferred_element_type=jnp.float32)


def _dot(a, b):
    return jnp.dot(a, b, preferred_element_type=jnp.float32)


def _knn_onehots(d2, coli_f):
    ohs = []
    for _ in range(_K):
        m = jnp.min(d2, axis=1, keepdims=True)
        cand = jnp.where(d2 <= m, coli_f, 3.0 * _NP)
        jstar = jnp.min(cand, axis=1, keepdims=True)
        ohb = coli_f == jstar
        ohs.append(ohb.astype(jnp.float32))
        d2 = jnp.where(ohb, _BIG, d2)
    return jnp.concatenate(ohs, axis=0)


def _mean_k(H, g):
    base = g * _K * _NP
    agg = H[base:base + _NP]
    for t in range(1, _K):
        agg = agg + H[base + t * _NP:base + (t + 1) * _NP]
    return agg * (1.0 / _K)


def _jet_kernel(x_ref, ccol_ref, crow_ref, s0_ref, b0_ref,
                WA1_ref, WB1_ref, b11_ref, W12_ref, b12_ref, W13_ref, b13_ref,
                Wsc1_ref, bsc1_ref,
                WA2_ref, WB2_ref, b21_ref, W22_ref, b22_ref, W23_ref, b23_ref,
                Wsc2_ref, bsc2_ref,
                Wf1_ref, Wf2_ref, bf_ref, Wg1_ref, bg1_ref, Wg2_ref, bg2_ref,
                out_ref):
    rowi = jax.lax.broadcasted_iota(jnp.int32, (_NP, _NP), 0)
    coli = jax.lax.broadcasted_iota(jnp.int32, (_NP, _NP), 1)
    coli_f = coli.astype(jnp.float32)
    invalid = (rowi == coli) | (coli >= _N)
    eye = (rowi == coli).astype(jnp.float32)
    rmask = jax.lax.broadcasted_iota(jnp.int32, (_NP, 1), 0) < _N

    F = x_ref.shape[2]
    x_all = jnp.reshape(x_ref[...], (_G * _NP, F))
    xbn = x_all * s0_ref[...] + b0_ref[...]

    A1 = _dot(xbn, WA1_ref[...]) + b11_ref[...]
    B1 = _dot(xbn, WB1_ref[...])
    sc1 = _dot(xbn, Wsc1_ref[...]) + bsc1_ref[...]

    h1_blocks = []
    for g in range(_G):
        cxr = crow_ref[g, 0:1, 0:_NP]
        cyr = crow_ref[g, 1:2, 0:_NP]
        cxc = ccol_ref[g, :, 0:1]
        cyc = ccol_ref[g, :, 1:2]
        dx = cxc - cxr
        dy = cyc - cyr
        d2 = jnp.where(invalid, _BIG, dx * dx + dy * dy)
        oh = _knn_onehots(d2, coli_f)
        Gg = _dot(oh, B1[g * _NP:(g + 1) * _NP])
        Ag = A1[g * _NP:(g + 1) * _NP]
        h1_blocks.append(jax.nn.relu(jnp.concatenate([Ag] * _K, 0) + Gg))
    H1 = jnp.concatenate(h1_blocks, axis=0)
    H2 = jax.nn.relu(_dot(H1, W12_ref[...]) + b12_ref[...])
    H3 = jax.nn.relu(_dot(H2, W13_ref[...]) + b13_ref[...])
    x1 = jax.nn.relu(
        jnp.concatenate([_mean_k(H3, g) for g in range(_G)], axis=0) + sc1)

    A2 = _dot(x1, WA2_ref[...]) + b21_ref[...]
    B2 = _dot(x1, WB2_ref[...])
    sc2 = _dot(x1, Wsc2_ref[...]) + bsc2_ref[...]

    h1_blocks = []
    for g in range(_G):
        x1g = x1[g * _NP:(g + 1) * _NP]
        gram = _dotT(x1g, x1g)
        diag_c = jnp.sum(x1g * x1g, axis=1, keepdims=True)
        diag_r = jnp.sum(gram * eye, axis=0, keepdims=True)
        d2 = diag_c - 2.0 * gram + diag_r
        d2 = jnp.where(invalid, _BIG, d2)
        oh = _knn_onehots(d2, coli_f)
        Gg = _dot(oh, B2[g * _NP:(g + 1) * _NP])
        Ag = A2[g * _NP:(g + 1) * _NP]
        h1_blocks.append(jax.nn.relu(jnp.concatenate([Ag] * _K, 0) + Gg))
    H1b = jnp.concatenate(h1_blocks, axis=0)
    H2b = jax.nn.relu(_dot(H1b, W22_ref[...]) + b22_ref[...])
    H3b = jax.nn.relu(_dot(H2b, W23_ref[...]) + b23_ref[...])
    x2 = jax.nn.relu(
        jnp.concatenate([_mean_k(H3b, g) for g in range(_G)], axis=0) + sc2)

    f = jax.nn.relu(_dot(x1, Wf1_ref[...]) + _dot(x2, Wf2_ref[...])
                    + bf_ref[...])
    pooled = jnp.concatenate(
        [jnp.sum(jnp.where(rmask, f[g * _NP:(g + 1) * _NP], 0.0),
                 axis=0, keepdims=True) for g in range(_G)], axis=0) / _N
    h = jax.nn.relu(_dot(pooled, Wg1_ref[...]) + bg1_ref[...])
    out = _dot(h, Wg2_ref[...]) + bg2_ref[...]
    out_ref[...] = jnp.reshape(out, (_G, 1, 128))


@jax.jit
def kernel(points, features, params):
    B, _, N = points.shape
    F = features.shape[1]

    g0, be0 = params['bn_fts']
    s0 = (g0 / jnp.sqrt(1.0 + _EPS)).reshape(1, F)
    b0 = be0.reshape(1, F)

    def conv_params(mlp, sc, fin):
        (W1, bb1, g1, be1), (W2, bb2, g2, be2), (W3, bb3, g3, be3) = mlp
        W1f, b1f = _fold_lin(W1, bb1, g1, be1)
        Wa, Wb = W1f[:, :fin], W1f[:, fin:]
        W2f, b2f = _fold_lin(W2, bb2, g2, be2)
        W3f, b3f = _fold_lin(W3, bb3, g3, be3)
        Wsc, gsc, bsc = sc
        Wscf = Wsc * (gsc / jnp.sqrt(1.0 + _EPS))[:, None]
        h = W1f.shape[0]
        return [(Wa - Wb).T, Wb.T, b1f.reshape(1, h),
                W2f.T, b2f.reshape(1, h), W3f.T, b3f.reshape(1, h),
                Wscf.T, bsc.reshape(1, h)]

    p1 = conv_params(params['conv1_mlp'], params['conv1_sc'], F)
    p2 = conv_params(params['conv2_mlp'], params['conv2_sc'], 32)

    Wf, gf, bf = params['fusion']
    Wff = (Wf * (gf / jnp.sqrt(1.0 + _EPS))[:, None]).T
    Wf1, Wf2 = Wff[:32], Wff[32:]
    bff = bf.reshape(1, -1)

    Wg1, bg1 = params['fc1']
    Wg1T = Wg1.T
    bg1 = bg1.reshape(1, -1)
    Wg2, bg2 = params['fc2']
    ncls = Wg2.shape[0]
    Wg2T = jnp.pad(Wg2.T, ((0, 0), (0, 128 - ncls)))
    bg2p = jnp.pad(bg2.reshape(1, -1), ((0, 0), (0, 128 - ncls)))

    xT = jnp.transpose(features, (0, 2, 1))
    xT = jnp.pad(xT, ((0, 0), (0, _NP - N), (0, 0)))
    cT = jnp.transpose(points, (0, 2, 1))
    ccol = jnp.pad(cT, ((0, 0), (0, _NP - N), (0, F - 2)))
    crow = jnp.pad(points, ((0, 0), (0, 8 - 2), (0, F - N)))

    weights = [s0, b0] + p1 + p2 + [Wf1, Wf2, bff, Wg1T, bg1, Wg2T, bg2p]

    def wspec(w):
        nd = w.ndim
        return pl.BlockSpec(w.shape, lambda i, _nd=nd: (0,) * _nd)

    out = pl.pallas_call(
        _jet_kernel,
        grid=(B // _G,),
        in_specs=[pl.BlockSpec((_G, _NP, F), lambda i: (i, 0, 0)),
                  pl.BlockSpec((_G, _NP, F), lambda i: (i, 0, 0)),
                  pl.BlockSpec((_G, 8, F), lambda i: (i, 0, 0))]
                 + [wspec(w) for w in weights],
        out_specs=pl.BlockSpec((_G, 1, 128), lambda i: (i, 0, 0)),
        out_shape=jax.ShapeDtypeStruct((B, 1, 128), jnp.float32),
    )(xT, ccol, crow, *weights)
    return out[:, 0, :ncls]


# speedup vs baseline: 26.3116x; 1.2898x over previous
import jax
import jax.numpy as jnp
from jax.experimental import pallas as pl

_EPS = 1e-5
_N = 100
_NP = 104
_K = 7
_G = 8
_BIG = 1e10


def _fold_lin(W, bb, g, be):
    s = g * (1.0 / jnp.sqrt(1.0 + _EPS))
    return W * s[:, None], bb * s + be


def _dotT(a, b):
    return jax.lax.dot_general(a, b, (((1,), (1,)), ((), ())),
                               preferred_element_type=jnp.float32)


def _dot(a, b):
    return jnp.dot(a, b, preferred_element_type=jnp.float32)


def _knn_onehots(d2):
    ohs = []
    for _ in range(_K):
        m = jnp.min(d2, axis=1, keepdims=True)
        ohb = d2 <= m
        ohs.append(ohb.astype(jnp.float32))
        d2 = jnp.where(ohb, _BIG, d2)
    return jnp.concatenate(ohs, axis=0)


def _mean_k(H, g):
    base = g * _K * _NP
    agg = H[base:base + _NP]
    for t in range(1, _K):
        agg = agg + H[base + t * _NP:base + (t + 1) * _NP]
    return agg * (1.0 / _K)


def _jet_kernel(x_ref, ccol_ref, crow_ref, s0_ref, b0_ref,
                WA1_ref, WB1_ref, b11_ref, W12_ref, b12_ref, W13_ref, b13_ref,
                Wsc1_ref, bsc1_ref,
                WA2_ref, WB2_ref, b21_ref, W22_ref, b22_ref, W23_ref, b23_ref,
                Wsc2_ref, bsc2_ref,
                Wf1_ref, Wf2_ref, bf_ref, Wg1_ref, bg1_ref, Wg2_ref, bg2_ref,
                out_ref):
    rowi = jax.lax.broadcasted_iota(jnp.int32, (_NP, _NP), 0)
    coli = jax.lax.broadcasted_iota(jnp.int32, (_NP, _NP), 1)
    invalid = (rowi == coli) | (coli >= _N)
    eye = (rowi == coli).astype(jnp.float32)
    rmask = jax.lax.broadcasted_iota(jnp.int32, (_NP, 1), 0) < _N

    F = x_ref.shape[2]
    x_all = jnp.reshape(x_ref[...], (_G * _NP, F))
    xbn = x_all * s0_ref[...] + b0_ref[...]

    A1 = _dot(xbn, WA1_ref[...]) + b11_ref[...]
    B1 = _dot(xbn, WB1_ref[...])
    sc1 = _dot(xbn, Wsc1_ref[...]) + bsc1_ref[...]

    h1_blocks = []
    for g in range(_G):
        cxr = crow_ref[g, 0:1, 0:_NP]
        cyr = crow_ref[g, 1:2, 0:_NP]
        cxc = ccol_ref[g, :, 0:1]
        cyc = ccol_ref[g, :, 1:2]
        dx = cxc - cxr
        dy = cyc - cyr
        d2 = jnp.where(invalid, _BIG, dx * dx + dy * dy)
        oh = _knn_onehots(d2)
        Gg = _dot(oh, B1[g * _NP:(g + 1) * _NP])
        Ag = A1[g * _NP:(g + 1) * _NP]
        h1_blocks.append(jax.nn.relu(jnp.concatenate([Ag] * _K, 0) + Gg))
    H1 = jnp.concatenate(h1_blocks, axis=0)
    H2 = jax.nn.relu(_dot(H1, W12_ref[...]) + b12_ref[...])
    H3 = jax.nn.relu(_dot(H2, W13_ref[...]) + b13_ref[...])
    x1 = jax.nn.relu(
        jnp.concatenate([_mean_k(H3, g) for g in range(_G)], axis=0) + sc1)

    A2 = _dot(x1, WA2_ref[...]) + b21_ref[...]
    B2 = _dot(x1, WB2_ref[...])
    sc2 = _dot(x1, Wsc2_ref[...]) + bsc2_ref[...]

    h1_blocks = []
    for g in range(_G):
        x1g = x1[g * _NP:(g + 1) * _NP]
        gram = _dotT(x1g, x1g)
        diag_c = jnp.sum(x1g * x1g, axis=1, keepdims=True)
        diag_r = jnp.sum(gram * eye, axis=0, keepdims=True)
        d2 = diag_c - 2.0 * gram + diag_r
        d2 = jnp.where(invalid, _BIG, d2)
        oh = _knn_onehots(d2)
        Gg = _dot(oh, B2[g * _NP:(g + 1) * _NP])
        Ag = A2[g * _NP:(g + 1) * _NP]
        h1_blocks.append(jax.nn.relu(jnp.concatenate([Ag] * _K, 0) + Gg))
    H1b = jnp.concatenate(h1_blocks, axis=0)
    H2b = jax.nn.relu(_dot(H1b, W22_ref[...]) + b22_ref[...])
    H3b = jax.nn.relu(_dot(H2b, W23_ref[...]) + b23_ref[...])
    x2 = jax.nn.relu(
        jnp.concatenate([_mean_k(H3b, g) for g in range(_G)], axis=0) + sc2)

    f = jax.nn.relu(_dot(x1, Wf1_ref[...]) + _dot(x2, Wf2_ref[...])
                    + bf_ref[...])
    pooled = jnp.concatenate(
        [jnp.sum(jnp.where(rmask, f[g * _NP:(g + 1) * _NP], 0.0),
                 axis=0, keepdims=True) for g in range(_G)], axis=0) / _N
    h = jax.nn.relu(_dot(pooled, Wg1_ref[...]) + bg1_ref[...])
    out = _dot(h, Wg2_ref[...]) + bg2_ref[...]
    out_ref[...] = jnp.reshape(out, (_G, 1, 128))


@jax.jit
def kernel(points, features, params):
    B, _, N = points.shape
    F = features.shape[1]

    g0, be0 = params['bn_fts']
    s0 = (g0 / jnp.sqrt(1.0 + _EPS)).reshape(1, F)
    b0 = be0.reshape(1, F)

    def conv_params(mlp, sc, fin):
        (W1, bb1, g1, be1), (W2, bb2, g2, be2), (W3, bb3, g3, be3) = mlp
        W1f, b1f = _fold_lin(W1, bb1, g1, be1)
        Wa, Wb = W1f[:, :fin], W1f[:, fin:]
        W2f, b2f = _fold_lin(W2, bb2, g2, be2)
        W3f, b3f = _fold_lin(W3, bb3, g3, be3)
        Wsc, gsc, bsc = sc
        Wscf = Wsc * (gsc / jnp.sqrt(1.0 + _EPS))[:, None]
        h = W1f.shape[0]
        return [(Wa - Wb).T, Wb.T, b1f.reshape(1, h),
                W2f.T, b2f.reshape(1, h), W3f.T, b3f.reshape(1, h),
                Wscf.T, bsc.reshape(1, h)]

    p1 = conv_params(params['conv1_mlp'], params['conv1_sc'], F)
    p2 = conv_params(params['conv2_mlp'], params['conv2_sc'], 32)

    Wf, gf, bf = params['fusion']
    Wff = (Wf * (gf / jnp.sqrt(1.0 + _EPS))[:, None]).T
    Wf1, Wf2 = Wff[:32], Wff[32:]
    bff = bf.reshape(1, -1)

    Wg1, bg1 = params['fc1']
    Wg1T = Wg1.T
    bg1 = bg1.reshape(1, -1)
    Wg2, bg2 = params['fc2']
    ncls = Wg2.shape[0]
    Wg2T = jnp.pad(Wg2.T, ((0, 0), (0, 128 - ncls)))
    bg2p = jnp.pad(bg2.reshape(1, -1), ((0, 0), (0, 128 - ncls)))

    xT = jnp.transpose(features, (0, 2, 1))
    xT = jnp.pad(xT, ((0, 0), (0, _NP - N), (0, 0)))
    cT = jnp.transpose(points, (0, 2, 1))
    ccol = jnp.pad(cT, ((0, 0), (0, _NP - N), (0, F - 2)))
    crow = jnp.pad(points, ((0, 0), (0, 8 - 2), (0, F - N)))

    weights = [s0, b0] + p1 + p2 + [Wf1, Wf2, bff, Wg1T, bg1, Wg2T, bg2p]

    def wspec(w):
        nd = w.ndim
        return pl.BlockSpec(w.shape, lambda i, _nd=nd: (0,) * _nd)

    out = pl.pallas_call(
        _jet_kernel,
        grid=(B // _G,),
        in_specs=[pl.BlockSpec((_G, _NP, F), lambda i: (i, 0, 0)),
                  pl.BlockSpec((_G, _NP, F), lambda i: (i, 0, 0)),
                  pl.BlockSpec((_G, 8, F), lambda i: (i, 0, 0))]
                 + [wspec(w) for w in weights],
        out_specs=pl.BlockSpec((_G, 1, 128), lambda i: (i, 0, 0)),
        out_shape=jax.ShapeDtypeStruct((B, 1, 128), jnp.float32),
    )(xT, ccol, crow, *weights)
    return out[:, 0, :ncls]


# speedup vs baseline: 27.6706x; 1.0517x over previous
import jax
import jax.numpy as jnp
from jax.experimental import pallas as pl

_EPS = 1e-5
_N = 100
_NP = 104
_K = 7
_G = 16
_BIG = 1e10


def _fold_lin(W, bb, g, be):
    s = g * (1.0 / jnp.sqrt(1.0 + _EPS))
    return W * s[:, None], bb * s + be


def _dotT(a, b):
    return jax.lax.dot_general(a, b, (((1,), (1,)), ((), ())),
                               preferred_element_type=jnp.float32)


def _dot(a, b):
    return jnp.dot(a, b, preferred_element_type=jnp.float32)


def _knn_onehots(d2):
    ohs = []
    for _ in range(_K):
        m = jnp.min(d2, axis=1, keepdims=True)
        ohb = d2 <= m
        ohs.append(ohb.astype(jnp.float32))
        d2 = jnp.where(ohb, _BIG, d2)
    return jnp.concatenate(ohs, axis=0)


def _mean_k(H, g):
    base = g * _K * _NP
    agg = H[base:base + _NP]
    for t in range(1, _K):
        agg = agg + H[base + t * _NP:base + (t + 1) * _NP]
    return agg * (1.0 / _K)


def _jet_kernel(x_ref, ccol_ref, crow_ref, s0_ref, b0_ref,
                WA1_ref, WB1_ref, b11_ref, W12_ref, b12_ref, W13_ref, b13_ref,
                Wsc1_ref, bsc1_ref,
                WA2_ref, WB2_ref, b21_ref, W22_ref, b22_ref, W23_ref, b23_ref,
                Wsc2_ref, bsc2_ref,
                Wf1_ref, Wf2_ref, bf_ref, Wg1_ref, bg1_ref, Wg2_ref, bg2_ref,
                out_ref):
    rowi = jax.lax.broadcasted_iota(jnp.int32, (_NP, _NP), 0)
    coli = jax.lax.broadcasted_iota(jnp.int32, (_NP, _NP), 1)
    invalid = (rowi == coli) | (coli >= _N)
    eye = (rowi == coli).astype(jnp.float32)
    rmask = jax.lax.broadcasted_iota(jnp.int32, (_NP, 1), 0) < _N

    F = x_ref.shape[2]
    x_all = jnp.reshape(x_ref[...], (_G * _NP, F))
    xbn = x_all * s0_ref[...] + b0_ref[...]

    A1 = _dot(xbn, WA1_ref[...]) + b11_ref[...]
    B1 = _dot(xbn, WB1_ref[...])
    sc1 = _dot(xbn, Wsc1_ref[...]) + bsc1_ref[...]

    h1_blocks = []
    for g in range(_G):
        cxr = crow_ref[g, 0:1, 0:_NP]
        cyr = crow_ref[g, 1:2, 0:_NP]
        cxc = ccol_ref[g, :, 0:1]
        cyc = ccol_ref[g, :, 1:2]
        dx = cxc - cxr
        dy = cyc - cyr
        d2 = jnp.where(invalid, _BIG, dx * dx + dy * dy)
        oh = _knn_onehots(d2)
        Gg = _dot(oh, B1[g * _NP:(g + 1) * _NP])
        Ag = A1[g * _NP:(g + 1) * _NP]
        h1_blocks.append(jax.nn.relu(jnp.concatenate([Ag] * _K, 0) + Gg))
    H1 = jnp.concatenate(h1_blocks, axis=0)
    H2 = jax.nn.relu(_dot(H1, W12_ref[...]) + b12_ref[...])
    H3 = jax.nn.relu(_dot(H2, W13_ref[...]) + b13_ref[...])
    x1 = jax.nn.relu(
        jnp.concatenate([_mean_k(H3, g) for g in range(_G)], axis=0) + sc1)

    A2 = _dot(x1, WA2_ref[...]) + b21_ref[...]
    B2 = _dot(x1, WB2_ref[...])
    sc2 = _dot(x1, Wsc2_ref[...]) + bsc2_ref[...]

    h1_blocks = []
    for g in range(_G):
        x1g = x1[g * _NP:(g + 1) * _NP]
        gram = _dotT(x1g, x1g)
        diag_c = jnp.sum(x1g * x1g, axis=1, keepdims=True)
        diag_r = jnp.sum(gram * eye, axis=0, keepdims=True)
        d2 = diag_c - 2.0 * gram + diag_r
        d2 = jnp.where(invalid, _BIG, d2)
        oh = _knn_onehots(d2)
        Gg = _dot(oh, B2[g * _NP:(g + 1) * _NP])
        Ag = A2[g * _NP:(g + 1) * _NP]
        h1_blocks.append(jax.nn.relu(jnp.concatenate([Ag] * _K, 0) + Gg))
    H1b = jnp.concatenate(h1_blocks, axis=0)
    H2b = jax.nn.relu(_dot(H1b, W22_ref[...]) + b22_ref[...])
    H3b = jax.nn.relu(_dot(H2b, W23_ref[...]) + b23_ref[...])
    x2 = jax.nn.relu(
        jnp.concatenate([_mean_k(H3b, g) for g in range(_G)], axis=0) + sc2)

    f = jax.nn.relu(_dot(x1, Wf1_ref[...]) + _dot(x2, Wf2_ref[...])
                    + bf_ref[...])
    pooled = jnp.concatenate(
        [jnp.sum(jnp.where(rmask, f[g * _NP:(g + 1) * _NP], 0.0),
                 axis=0, keepdims=True) for g in range(_G)], axis=0) / _N
    h = jax.nn.relu(_dot(pooled, Wg1_ref[...]) + bg1_ref[...])
    out = _dot(h, Wg2_ref[...]) + bg2_ref[...]
    out_ref[...] = jnp.reshape(out, (_G, 1, 128))


@jax.jit
def kernel(points, features, params):
    B, _, N = points.shape
    F = features.shape[1]

    g0, be0 = params['bn_fts']
    s0 = (g0 / jnp.sqrt(1.0 + _EPS)).reshape(1, F)
    b0 = be0.reshape(1, F)

    def conv_params(mlp, sc, fin):
        (W1, bb1, g1, be1), (W2, bb2, g2, be2), (W3, bb3, g3, be3) = mlp
        W1f, b1f = _fold_lin(W1, bb1, g1, be1)
        Wa, Wb = W1f[:, :fin], W1f[:, fin:]
        W2f, b2f = _fold_lin(W2, bb2, g2, be2)
        W3f, b3f = _fold_lin(W3, bb3, g3, be3)
        Wsc, gsc, bsc = sc
        Wscf = Wsc * (gsc / jnp.sqrt(1.0 + _EPS))[:, None]
        h = W1f.shape[0]
        return [(Wa - Wb).T, Wb.T, b1f.reshape(1, h),
                W2f.T, b2f.reshape(1, h), W3f.T, b3f.reshape(1, h),
                Wscf.T, bsc.reshape(1, h)]

    p1 = conv_params(params['conv1_mlp'], params['conv1_sc'], F)
    p2 = conv_params(params['conv2_mlp'], params['conv2_sc'], 32)

    Wf, gf, bf = params['fusion']
    Wff = (Wf * (gf / jnp.sqrt(1.0 + _EPS))[:, None]).T
    Wf1, Wf2 = Wff[:32], Wff[32:]
    bff = bf.reshape(1, -1)

    Wg1, bg1 = params['fc1']
    Wg1T = Wg1.T
    bg1 = bg1.reshape(1, -1)
    Wg2, bg2 = params['fc2']
    ncls = Wg2.shape[0]
    Wg2T = jnp.pad(Wg2.T, ((0, 0), (0, 128 - ncls)))
    bg2p = jnp.pad(bg2.reshape(1, -1), ((0, 0), (0, 128 - ncls)))

    xT = jnp.transpose(features, (0, 2, 1))
    xT = jnp.pad(xT, ((0, 0), (0, _NP - N), (0, 0)))
    cT = jnp.transpose(points, (0, 2, 1))
    ccol = jnp.pad(cT, ((0, 0), (0, _NP - N), (0, F - 2)))
    crow = jnp.pad(points, ((0, 0), (0, 8 - 2), (0, F - N)))

    weights = [s0, b0] + p1 + p2 + [Wf1, Wf2, bff, Wg1T, bg1, Wg2T, bg2p]

    def wspec(w):
        nd = w.ndim
        return pl.BlockSpec(w.shape, lambda i, _nd=nd: (0,) * _nd)

    out = pl.pallas_call(
        _jet_kernel,
        grid=(B // _G,),
        in_specs=[pl.BlockSpec((_G, _NP, F), lambda i: (i, 0, 0)),
                  pl.BlockSpec((_G, _NP, F), lambda i: (i, 0, 0)),
                  pl.BlockSpec((_G, 8, F), lambda i: (i, 0, 0))]
                 + [wspec(w) for w in weights],
        out_specs=pl.BlockSpec((_G, 1, 128), lambda i: (i, 0, 0)),
        out_shape=jax.ShapeDtypeStruct((B, 1, 128), jnp.float32),
    )(xT, ccol, crow, *weights)
    return out[:, 0, :ncls]
